# baseline (speedup 1.0000x reference)
import functools

import jax
import jax.numpy as jnp
from jax import lax
from jax.experimental import pallas as pl
from jax.experimental.pallas import tpu as pltpu
from jax.experimental.pallas import tpu_sc as plsc

N_NODES = 10000
N_EDGES = 320000
NPAD = 10240
NC = 2
NS = 16
NW = NC * NS
E_PER_TILE = N_EDGES // NW
CHUNK = NPAD // NS
LANES = 16
E_SPLIT = 4992


def _sc_body(x0_hbm, ei_hbm, attr_hbm, out_hbm,
             x0_v, src_v, dst_v, attr_v, acc_v, red_v, shared):
    c = lax.axis_index("c")
    s = lax.axis_index("s")
    wid = c * NS + s
    base = wid * E_PER_TILE

    pltpu.sync_copy(x0_hbm, x0_v)
    pltpu.sync_copy(ei_hbm.at[jnp.int32(0), pl.ds(base, E_PER_TILE)], src_v)
    pltpu.sync_copy(ei_hbm.at[jnp.int32(1), pl.ds(base, E_PER_TILE)], dst_v)

    zeros16 = jnp.zeros((LANES,), jnp.float32)

    def zbody(j, carry):
        off = j * LANES
        acc_v[pl.ds(off, LANES)] = zeros16
        return carry

    lax.fori_loop(jnp.int32(0), jnp.int32(NPAD // LANES), zbody, None)

    lane = lax.iota(jnp.int32, 16)
    col0 = jnp.zeros((LANES,), jnp.int32)
    col1 = jnp.ones((LANES,), jnp.int32)

    for e_off, e_cnt in ((0, E_SPLIT), (E_SPLIT, E_PER_TILE - E_SPLIT)):
        pltpu.sync_copy(attr_hbm.at[pl.ds(base + e_off, e_cnt), :],
                        attr_v.at[pl.ds(jnp.int32(0), e_cnt), :])

        def ebody(j, carry, e_off=e_off):
            off = j * LANES
            abs_off = off + e_off
            srcv = src_v[pl.ds(abs_off, LANES)]
            dstv = dst_v[pl.ds(abs_off, LANES)]
            xs = plsc.load_gather(x0_v, [srcv])
            xd = plsc.load_gather(x0_v, [dstv])
            eids = lane + off
            w0 = plsc.load_gather(attr_v, [eids, col0])
            w1 = plsc.load_gather(attr_v, [eids, col1])
            contrib = (xs - xd) * (w0 + w1)
            plsc.addupdate_scatter(acc_v, [dstv], contrib)
            return carry

        lax.fori_loop(jnp.int32(0), jnp.int32(e_cnt // LANES), ebody, None)

    pltpu.sync_copy(acc_v, shared.at[s])
    plsc.subcore_barrier()

    nbase = s * CHUNK
    for r in range(NS):
        pltpu.sync_copy(shared.at[jnp.int32(r), pl.ds(nbase, CHUNK)],
                        red_v.at[jnp.int32(r)])

    def rbody(j, carry):
        off = j * LANES
        a = red_v[jnp.int32(0), pl.ds(off, LANES)]
        for r in range(1, NS):
            a = a + red_v[jnp.int32(r), pl.ds(off, LANES)]
        acc_v[pl.ds(off, LANES)] = a
        return carry

    lax.fori_loop(jnp.int32(0), jnp.int32(CHUNK // LANES), rbody, None)
    pltpu.sync_copy(acc_v.at[pl.ds(jnp.int32(0), CHUNK)],
                    out_hbm.at[c, pl.ds(nbase, CHUNK)])


@jax.jit
def _sc_call(x0, ei, attr):
    mesh = plsc.VectorSubcoreMesh(core_axis_name="c", subcore_axis_name="s")
    return pl.kernel(
        _sc_body,
        out_type=jax.ShapeDtypeStruct((NC, NPAD), jnp.float32),
        mesh=mesh,
        compiler_params=pltpu.CompilerParams(
            needs_layout_passes=False, use_tc_tiling_on_sc=False),
        scratch_types=[
            pltpu.VMEM((N_NODES,), jnp.float32),
            pltpu.VMEM((E_PER_TILE,), jnp.int32),
            pltpu.VMEM((E_PER_TILE,), jnp.int32),
            pltpu.VMEM((E_PER_TILE - E_SPLIT, 4), jnp.float32),

            pltpu.VMEM((NPAD,), jnp.float32),
            pltpu.VMEM((NS, CHUNK), jnp.float32),
            pltpu.VMEM_SHARED((NS, NPAD), jnp.float32),
        ],
    )(x0, ei, attr)


def _combine_body(p_ref, o_ref):
    o_ref[...] = p_ref[0, :] + p_ref[1, :]


@jax.jit
def _combine(partials):
    return pl.pallas_call(
        _combine_body,
        out_shape=jax.ShapeDtypeStruct((NPAD,), jnp.float32),
    )(partials)


def kernel(x, edge_index, edge_attr):
    x0 = x[:, 0]
    ei = edge_index.astype(jnp.int32)
    partials = _sc_call(x0, ei, edge_attr)
    return _combine(partials)[:N_NODES]


# speedup vs baseline: 4.5945x; 4.5945x over previous
import jax
import jax.numpy as jnp
from jax import lax
from jax.experimental import pallas as pl
from jax.experimental.pallas import tpu as pltpu
from jax.experimental.pallas import tpu_sc as plsc

N_NODES = 10000
N_EDGES = 320000
NPAD = 10240
NC = 2
NS = 16
NW = NC * NS
E_PER_TILE = N_EDGES // NW
CHUNK = NPAD // NS
LANES = 16


def _sc_body(x0_hbm, src_hbm, dst_hbm, w_hbm, out_hbm,
             x0_v, src_v, dst_v, w_v, acc_v, red_v, shared):
    c = lax.axis_index("c")
    s = lax.axis_index("s")
    wid = c * NS + s
    base = wid * E_PER_TILE

    pltpu.sync_copy(x0_hbm, x0_v)
    pltpu.sync_copy(src_hbm.at[pl.ds(base, E_PER_TILE)], src_v)
    pltpu.sync_copy(dst_hbm.at[pl.ds(base, E_PER_TILE)], dst_v)
    pltpu.sync_copy(w_hbm.at[pl.ds(base, E_PER_TILE)], w_v)

    zeros16 = jnp.zeros((LANES,), jnp.float32)

    def zbody(j, carry):
        acc_v[pl.ds(j * LANES, LANES)] = zeros16
        return carry

    lax.fori_loop(jnp.int32(0), jnp.int32(NPAD // LANES), zbody, None)

    def ebody(j, carry):
        off = j * LANES
        srcv = src_v[pl.ds(off, LANES)]
        dstv = dst_v[pl.ds(off, LANES)]
        xs = plsc.load_gather(x0_v, [srcv])
        xd = plsc.load_gather(x0_v, [dstv])
        wv = w_v[pl.ds(off, LANES)]
        contrib = (xs - xd) * wv
        plsc.addupdate_scatter(acc_v, [dstv], contrib)
        return carry

    lax.fori_loop(jnp.int32(0), jnp.int32(E_PER_TILE // LANES), ebody, None)

    pltpu.sync_copy(acc_v, shared.at[s])
    plsc.subcore_barrier()

    nbase = s * CHUNK
    for r in range(NS):
        pltpu.sync_copy(shared.at[jnp.int32(r), pl.ds(nbase, CHUNK)],
                        red_v.at[jnp.int32(r)])

    def rbody(j, carry):
        off = j * LANES
        a = red_v[jnp.int32(0), pl.ds(off, LANES)]
        for r in range(1, NS):
            a = a + red_v[jnp.int32(r), pl.ds(off, LANES)]
        acc_v[pl.ds(off, LANES)] = a
        return carry

    lax.fori_loop(jnp.int32(0), jnp.int32(CHUNK // LANES), rbody, None)
    pltpu.sync_copy(acc_v.at[pl.ds(jnp.int32(0), CHUNK)],
                    out_hbm.at[c, pl.ds(nbase, CHUNK)])


@jax.jit
def _sc_call(x0, src, dst, w):
    mesh = plsc.VectorSubcoreMesh(core_axis_name="c", subcore_axis_name="s")
    return pl.kernel(
        _sc_body,
        out_type=jax.ShapeDtypeStruct((NC, NPAD), jnp.float32),
        mesh=mesh,
        compiler_params=pltpu.CompilerParams(
            needs_layout_passes=False, use_tc_tiling_on_sc=False),
        scratch_types=[
            pltpu.VMEM((N_NODES,), jnp.float32),
            pltpu.VMEM((E_PER_TILE,), jnp.int32),
            pltpu.VMEM((E_PER_TILE,), jnp.int32),
            pltpu.VMEM((E_PER_TILE,), jnp.float32),
            pltpu.VMEM((NPAD,), jnp.float32),
            pltpu.VMEM((NS, CHUNK), jnp.float32),
            pltpu.VMEM_SHARED((NS, NPAD), jnp.float32),
        ],
    )(x0, src, dst, w)


def _combine_body(p_ref, o_ref):
    o_ref[...] = p_ref[0, :] + p_ref[1, :]


@jax.jit
def _combine(partials):
    return pl.pallas_call(
        _combine_body,
        out_shape=jax.ShapeDtypeStruct((NPAD,), jnp.float32),
    )(partials)


def kernel(x, edge_index, edge_attr):
    x0 = x[:, 0]
    src = edge_index[0].astype(jnp.int32)
    dst = edge_index[1].astype(jnp.int32)
    w = edge_attr[:, 0] + edge_attr[:, 1]
    partials = _sc_call(x0, src, dst, w)
    return _combine(partials)[:N_NODES]


# speedup vs baseline: 4.7150x; 1.0262x over previous
import jax
import jax.numpy as jnp
from jax import lax
from jax.experimental import pallas as pl
from jax.experimental.pallas import tpu as pltpu
from jax.experimental.pallas import tpu_sc as plsc

N_NODES = 10000
N_EDGES = 320000
NPAD = 10240
NC = 2
NS = 16
NW = NC * NS
E_PER_TILE = N_EDGES // NW
CHUNK = NPAD // NS
LANES = 16


def _sc_body(x0_hbm, src_hbm, dst_hbm, w_hbm, out_hbm,
             x0_v, src_v, dst_v, w_v, acc_v, red_v, shared):
    c = lax.axis_index("c")
    s = lax.axis_index("s")
    wid = c * NS + s
    base = wid * E_PER_TILE

    pltpu.sync_copy(x0_hbm, x0_v)
    pltpu.sync_copy(src_hbm.at[pl.ds(base, E_PER_TILE)], src_v)
    pltpu.sync_copy(dst_hbm.at[pl.ds(base, E_PER_TILE)], dst_v)
    pltpu.sync_copy(w_hbm.at[pl.ds(base, E_PER_TILE)], w_v)

    zeros16 = jnp.zeros((LANES,), jnp.float32)

    ZUNROLL = 8

    def zbody(j, carry):
        boff = j * (LANES * ZUNROLL)
        for u in range(ZUNROLL):
            acc_v[pl.ds(boff + u * LANES, LANES)] = zeros16
        return carry

    lax.fori_loop(jnp.int32(0), jnp.int32(NPAD // (LANES * ZUNROLL)),
                  zbody, None)

    EUNROLL = 5

    def ebody(j, carry):
        boff = j * (LANES * EUNROLL)
        for u in range(EUNROLL):
            off = boff + u * LANES
            srcv = src_v[pl.ds(off, LANES)]
            dstv = dst_v[pl.ds(off, LANES)]
            xs = plsc.load_gather(x0_v, [srcv])
            xd = plsc.load_gather(x0_v, [dstv])
            wv = w_v[pl.ds(off, LANES)]
            contrib = (xs - xd) * wv
            plsc.addupdate_scatter(acc_v, [dstv], contrib)
        return carry

    lax.fori_loop(jnp.int32(0), jnp.int32(E_PER_TILE // (LANES * EUNROLL)),
                  ebody, None)

    pltpu.sync_copy(acc_v, shared.at[s])
    plsc.subcore_barrier()

    nbase = s * CHUNK
    for r in range(NS):
        pltpu.sync_copy(shared.at[jnp.int32(r), pl.ds(nbase, CHUNK)],
                        red_v.at[jnp.int32(r)])

    def rbody(j, carry):
        off = j * LANES
        a = red_v[jnp.int32(0), pl.ds(off, LANES)]
        for r in range(1, NS):
            a = a + red_v[jnp.int32(r), pl.ds(off, LANES)]
        acc_v[pl.ds(off, LANES)] = a
        return carry

    lax.fori_loop(jnp.int32(0), jnp.int32(CHUNK // LANES), rbody, None)
    pltpu.sync_copy(acc_v.at[pl.ds(jnp.int32(0), CHUNK)],
                    out_hbm.at[c, pl.ds(nbase, CHUNK)])


@jax.jit
def _sc_call(x0, src, dst, w):
    mesh = plsc.VectorSubcoreMesh(core_axis_name="c", subcore_axis_name="s")
    return pl.kernel(
        _sc_body,
        out_type=jax.ShapeDtypeStruct((NC, NPAD), jnp.float32),
        mesh=mesh,
        compiler_params=pltpu.CompilerParams(
            needs_layout_passes=False, use_tc_tiling_on_sc=False),
        scratch_types=[
            pltpu.VMEM((N_NODES,), jnp.float32),
            pltpu.VMEM((E_PER_TILE,), jnp.int32),
            pltpu.VMEM((E_PER_TILE,), jnp.int32),
            pltpu.VMEM((E_PER_TILE,), jnp.float32),
            pltpu.VMEM((NPAD,), jnp.float32),
            pltpu.VMEM((NS, CHUNK), jnp.float32),
            pltpu.VMEM_SHARED((NS, NPAD), jnp.float32),
        ],
    )(x0, src, dst, w)


def _combine_body(p_ref, o_ref):
    o_ref[...] = p_ref[0, :] + p_ref[1, :]


@jax.jit
def _combine(partials):
    return pl.pallas_call(
        _combine_body,
        out_shape=jax.ShapeDtypeStruct((NPAD,), jnp.float32),
    )(partials)


def kernel(x, edge_index, edge_attr):
    x0 = x[:, 0]
    src = edge_index[0].astype(jnp.int32)
    dst = edge_index[1].astype(jnp.int32)
    w = edge_attr[:, 0] + edge_attr[:, 1]
    partials = _sc_call(x0, src, dst, w)
    return _combine(partials)[:N_NODES]


# speedup vs baseline: 5.0940x; 1.0804x over previous
import jax
import jax.numpy as jnp
from jax import lax
from jax.experimental import pallas as pl
from jax.experimental.pallas import tpu as pltpu
from jax.experimental.pallas import tpu_sc as plsc

N_NODES = 10000
N_EDGES = 320000
NPAD = 10240
NC = 2
NS = 16
NW = NC * NS
E_PER_TILE = N_EDGES // NW
CHUNK = NPAD // NS
LANES = 16


def _sc_body(x0_hbm, src_hbm, dst_hbm, w_hbm, out_hbm,
             x0_v, src_v, dst_v, w_v, acc_v, red_v, shared):
    c = lax.axis_index("c")
    s = lax.axis_index("s")
    wid = c * NS + s
    base = wid * E_PER_TILE

    pltpu.sync_copy(x0_hbm, x0_v)
    pltpu.sync_copy(src_hbm.at[pl.ds(base, E_PER_TILE)], src_v)
    pltpu.sync_copy(dst_hbm.at[pl.ds(base, E_PER_TILE)], dst_v)
    pltpu.sync_copy(w_hbm.at[pl.ds(base, E_PER_TILE)], w_v)

    zeros16 = jnp.zeros((LANES,), jnp.float32)

    ZUNROLL = 8

    def zbody(j, carry):
        boff = j * (LANES * ZUNROLL)
        for u in range(ZUNROLL):
            acc_v[pl.ds(boff + u * LANES, LANES)] = zeros16
        return carry

    lax.fori_loop(jnp.int32(0), jnp.int32(NPAD // (LANES * ZUNROLL)),
                  zbody, None)

    @plsc.parallel_loop(jnp.int32(0), jnp.int32(E_PER_TILE // LANES),
                        jnp.int32(1), unroll=5)
    def ebody(j):
        off = j * LANES
        srcv = src_v[pl.ds(off, LANES)]
        dstv = dst_v[pl.ds(off, LANES)]
        xs = plsc.load_gather(x0_v, [srcv])
        xd = plsc.load_gather(x0_v, [dstv])
        wv = w_v[pl.ds(off, LANES)]
        contrib = (xs - xd) * wv
        plsc.addupdate_scatter(acc_v, [dstv], contrib)

    pltpu.sync_copy(acc_v, shared.at[s])
    plsc.subcore_barrier()

    nbase = s * CHUNK
    for r in range(NS):
        pltpu.sync_copy(shared.at[jnp.int32(r), pl.ds(nbase, CHUNK)],
                        red_v.at[jnp.int32(r)])

    def rbody(j, carry):
        off = j * LANES
        a = red_v[jnp.int32(0), pl.ds(off, LANES)]
        for r in range(1, NS):
            a = a + red_v[jnp.int32(r), pl.ds(off, LANES)]
        acc_v[pl.ds(off, LANES)] = a
        return carry

    lax.fori_loop(jnp.int32(0), jnp.int32(CHUNK // LANES), rbody, None)
    pltpu.sync_copy(acc_v.at[pl.ds(jnp.int32(0), CHUNK)],
                    out_hbm.at[c, pl.ds(nbase, CHUNK)])


@jax.jit
def _sc_call(x0, src, dst, w):
    mesh = plsc.VectorSubcoreMesh(core_axis_name="c", subcore_axis_name="s")
    return pl.kernel(
        _sc_body,
        out_type=jax.ShapeDtypeStruct((NC, NPAD), jnp.float32),
        mesh=mesh,
        compiler_params=pltpu.CompilerParams(
            needs_layout_passes=False, use_tc_tiling_on_sc=False),
        scratch_types=[
            pltpu.VMEM((N_NODES,), jnp.float32),
            pltpu.VMEM((E_PER_TILE,), jnp.int32),
            pltpu.VMEM((E_PER_TILE,), jnp.int32),
            pltpu.VMEM((E_PER_TILE,), jnp.float32),
            pltpu.VMEM((NPAD,), jnp.float32),
            pltpu.VMEM((NS, CHUNK), jnp.float32),
            pltpu.VMEM_SHARED((NS, NPAD), jnp.float32),
        ],
    )(x0, src, dst, w)


def _combine_body(p_ref, o_ref):
    o_ref[...] = p_ref[0, :] + p_ref[1, :]


@jax.jit
def _combine(partials):
    return pl.pallas_call(
        _combine_body,
        out_shape=jax.ShapeDtypeStruct((NPAD,), jnp.float32),
    )(partials)


def kernel(x, edge_index, edge_attr):
    x0 = x[:, 0]
    src = edge_index[0].astype(jnp.int32)
    dst = edge_index[1].astype(jnp.int32)
    w = edge_attr[:, 0] + edge_attr[:, 1]
    partials = _sc_call(x0, src, dst, w)
    return _combine(partials)[:N_NODES]


# speedup vs baseline: 5.2613x; 1.0328x over previous
import jax
import jax.numpy as jnp
from jax import lax
from jax.experimental import pallas as pl
from jax.experimental.pallas import tpu as pltpu
from jax.experimental.pallas import tpu_sc as plsc

N_NODES = 10000
N_EDGES = 320000
NPAD = 10240
NC = 2
NS = 16
NW = NC * NS
E_PER_TILE = N_EDGES // NW
CHUNK = NPAD // NS
LANES = 16


def _sc_body(x0_hbm, src_hbm, dst_hbm, w_hbm, out_hbm,
             x0_v, src_v, dst_v, w_v, acc_v, red_v, shared, sem):
    c = lax.axis_index("c")
    s = lax.axis_index("s")
    wid = c * NS + s
    base = wid * E_PER_TILE

    cps = [
        pltpu.async_copy(x0_hbm, x0_v, sem),
        pltpu.async_copy(src_hbm.at[pl.ds(base, E_PER_TILE)], src_v, sem),
        pltpu.async_copy(dst_hbm.at[pl.ds(base, E_PER_TILE)], dst_v, sem),
        pltpu.async_copy(w_hbm.at[pl.ds(base, E_PER_TILE)], w_v, sem),
    ]

    zeros16 = jnp.zeros((LANES,), jnp.float32)

    @plsc.parallel_loop(jnp.int32(0), jnp.int32(NPAD // LANES),
                        jnp.int32(1), unroll=8)
    def zbody(j):
        acc_v[pl.ds(j * LANES, LANES)] = zeros16

    for cp in cps:
        cp.wait()

    @plsc.parallel_loop(jnp.int32(0), jnp.int32(E_PER_TILE // LANES),
                        jnp.int32(1), unroll=5)
    def ebody(j):
        off = j * LANES
        srcv = src_v[pl.ds(off, LANES)]
        dstv = dst_v[pl.ds(off, LANES)]
        xs = plsc.load_gather(x0_v, [srcv])
        xd = plsc.load_gather(x0_v, [dstv])
        wv = w_v[pl.ds(off, LANES)]
        contrib = (xs - xd) * wv
        plsc.addupdate_scatter(acc_v, [dstv], contrib)

    pltpu.sync_copy(acc_v, shared.at[s])
    plsc.subcore_barrier()

    nbase = s * CHUNK
    for r in range(NS):
        pltpu.sync_copy(shared.at[jnp.int32(r), pl.ds(nbase, CHUNK)],
                        red_v.at[jnp.int32(r)])

    @plsc.parallel_loop(jnp.int32(0), jnp.int32(CHUNK // LANES),
                        jnp.int32(1), unroll=4)
    def rbody(j):
        off = j * LANES
        a = red_v[jnp.int32(0), pl.ds(off, LANES)]
        for r in range(1, NS):
            a = a + red_v[jnp.int32(r), pl.ds(off, LANES)]
        acc_v[pl.ds(off, LANES)] = a
    pltpu.sync_copy(acc_v.at[pl.ds(jnp.int32(0), CHUNK)],
                    out_hbm.at[c, pl.ds(nbase, CHUNK)])


@jax.jit
def _sc_call(x0, src, dst, w):
    mesh = plsc.VectorSubcoreMesh(core_axis_name="c", subcore_axis_name="s")
    return pl.kernel(
        _sc_body,
        out_type=jax.ShapeDtypeStruct((NC, NPAD), jnp.float32),
        mesh=mesh,
        compiler_params=pltpu.CompilerParams(
            needs_layout_passes=False, use_tc_tiling_on_sc=False),
        scratch_types=[
            pltpu.VMEM((N_NODES,), jnp.float32),
            pltpu.VMEM((E_PER_TILE,), jnp.int32),
            pltpu.VMEM((E_PER_TILE,), jnp.int32),
            pltpu.VMEM((E_PER_TILE,), jnp.float32),
            pltpu.VMEM((NPAD,), jnp.float32),
            pltpu.VMEM((NS, CHUNK), jnp.float32),
            pltpu.VMEM_SHARED((NS, NPAD), jnp.float32),
            pltpu.SemaphoreType.DMA,
        ],
    )(x0, src, dst, w)


def _combine_body(p_ref, o_ref):
    o_ref[...] = p_ref[0, :N_NODES] + p_ref[1, :N_NODES]


@jax.jit
def _combine(partials):
    return pl.pallas_call(
        _combine_body,
        out_shape=jax.ShapeDtypeStruct((N_NODES,), jnp.float32),
    )(partials)


def kernel(x, edge_index, edge_attr):
    x0 = x[:, 0]
    src = edge_index[0].astype(jnp.int32)
    dst = edge_index[1].astype(jnp.int32)
    w = edge_attr[:, 0] + edge_attr[:, 1]
    partials = _sc_call(x0, src, dst, w)
    return _combine(partials)


# speedup vs baseline: 5.3790x; 1.0224x over previous
import jax
import jax.numpy as jnp
from jax import lax
from jax.experimental import pallas as pl
from jax.experimental.pallas import tpu as pltpu
from jax.experimental.pallas import tpu_sc as plsc

N_NODES = 10000
N_EDGES = 320000
NPAD = 10240
NC = 2
NS = 16
NW = NC * NS
E_PER_TILE = N_EDGES // NW
CHUNK = NPAD // NS
LANES = 16


def _sc_body(x0_hbm, src_hbm, dst_hbm, w_hbm, out_hbm,
             x0_v, src_v, dst_v, w_v, acc_v, red_v, shared, sem):
    c = lax.axis_index("c")
    s = lax.axis_index("s")
    wid = c * NS + s
    base = wid * E_PER_TILE

    cps = [
        pltpu.async_copy(x0_hbm, x0_v, sem),
        pltpu.async_copy(src_hbm.at[pl.ds(base, E_PER_TILE)], src_v, sem),
        pltpu.async_copy(dst_hbm.at[pl.ds(base, E_PER_TILE)], dst_v, sem),
        pltpu.async_copy(w_hbm.at[pl.ds(base, E_PER_TILE)], w_v, sem),
    ]

    zeros16 = jnp.zeros((LANES,), jnp.float32)

    @plsc.parallel_loop(jnp.int32(0), jnp.int32(NPAD // LANES),
                        jnp.int32(1), unroll=8)
    def zbody(j):
        acc_v[pl.ds(j * LANES, LANES)] = zeros16

    for cp in cps:
        cp.wait()

    @plsc.parallel_loop(jnp.int32(0), jnp.int32(E_PER_TILE // LANES),
                        jnp.int32(1), unroll=8)
    def ebody(j):
        off = j * LANES
        srcv = src_v[pl.ds(off, LANES)]
        dstv = dst_v[pl.ds(off, LANES)]
        xs = plsc.load_gather(x0_v, [srcv])
        xd = plsc.load_gather(x0_v, [dstv])
        wv = w_v[pl.ds(off, LANES)]
        contrib = (xs - xd) * wv
        plsc.addupdate_scatter(acc_v, [dstv], contrib)

    pltpu.sync_copy(acc_v, shared.at[s])
    plsc.subcore_barrier()

    nbase = s * CHUNK
    for r in range(NS):
        pltpu.sync_copy(shared.at[jnp.int32(r), pl.ds(nbase, CHUNK)],
                        red_v.at[jnp.int32(r)])

    @plsc.parallel_loop(jnp.int32(0), jnp.int32(CHUNK // LANES),
                        jnp.int32(1), unroll=4)
    def rbody(j):
        off = j * LANES
        a = red_v[jnp.int32(0), pl.ds(off, LANES)]
        for r in range(1, NS):
            a = a + red_v[jnp.int32(r), pl.ds(off, LANES)]
        acc_v[pl.ds(off, LANES)] = a
    pltpu.sync_copy(acc_v.at[pl.ds(jnp.int32(0), CHUNK)],
                    out_hbm.at[pl.ds(c * NPAD + nbase, CHUNK)])


@jax.jit
def _sc_call(x0, src, dst, w):
    mesh = plsc.VectorSubcoreMesh(core_axis_name="c", subcore_axis_name="s")
    return pl.kernel(
        _sc_body,
        out_type=jax.ShapeDtypeStruct((NC * NPAD,), jnp.float32),
        mesh=mesh,
        compiler_params=pltpu.CompilerParams(
            needs_layout_passes=False, use_tc_tiling_on_sc=False),
        scratch_types=[
            pltpu.VMEM((N_NODES,), jnp.float32),
            pltpu.VMEM((E_PER_TILE,), jnp.int32),
            pltpu.VMEM((E_PER_TILE,), jnp.int32),
            pltpu.VMEM((E_PER_TILE,), jnp.float32),
            pltpu.VMEM((NPAD,), jnp.float32),
            pltpu.VMEM((NS, CHUNK), jnp.float32),
            pltpu.VMEM_SHARED((NS, NPAD), jnp.float32),
            pltpu.SemaphoreType.DMA,
        ],
    )(x0, src, dst, w)


def _combine_body(p_ref, o_ref):
    o_ref[...] = (p_ref[pl.ds(0, N_NODES)] +
                  p_ref[pl.ds(NPAD, N_NODES)])


@jax.jit
def _combine(partials):
    return pl.pallas_call(
        _combine_body,
        out_shape=jax.ShapeDtypeStruct((N_NODES,), jnp.float32),
    )(partials)


def kernel(x, edge_index, edge_attr):
    x0 = x[:, 0]
    src = edge_index[0].astype(jnp.int32)
    dst = edge_index[1].astype(jnp.int32)
    w = edge_attr[:, 0] + edge_attr[:, 1]
    partials = _sc_call(x0, src, dst, w)
    return _combine(partials)
